# baseline (speedup 1.0000x reference)
import functools

import jax
import jax.numpy as jnp
from jax import lax
from jax.experimental import pallas as pl
from jax.experimental.pallas import tpu as pltpu
from jax.experimental.pallas import tpu_sc as plsc

F32 = jnp.float32
NC, NS = 2, 16
K = 80
IBLK = 50
NGRP = 16


def _mesh():
    return plsc.VectorSubcoreMesh(core_axis_name="c", subcore_axis_name="s",
                                  num_cores=NC, num_subcores=NS)


def _pad16x8(n):
    return ((n + NS * 8 - 1) // (NS * 8)) * (NS * 8)


def _make_deg(N, E):
    TPE = E // (NC * NS)
    NCH = TPE // K
    NP = _pad16x8(N)
    NPT = NP // NS

    @functools.partial(
        pl.kernel,
        out_type=jax.ShapeDtypeStruct((NP, 128), F32),
        mesh=_mesh(),
        compiler_params=pltpu.CompilerParams(use_tc_tiling_on_sc=False),
        scratch_types=[
            pltpu.VMEM((K,), jnp.int32),
            pltpu.VMEM((K, 8), F32),
            pltpu.VMEM((NPT, 8), F32),
            pltpu.VMEM_SHARED((NP, 8), F32),
        ],
    )
    def deg_kernel(dst_hbm, zeros_hbm, ones_hbm, out_hbm,
                   idx_v, ones_v, wb_v, acc_sh):
        c = lax.axis_index("c")
        s = lax.axis_index("s")
        pltpu.sync_copy(ones_hbm, ones_v)
        pltpu.sync_copy(zeros_hbm, acc_sh.at[pl.ds(s * NPT, NPT)])
        plsc.subcore_barrier()
        base = (c * NS + s) * TPE

        def chunk(j, carry):
            pltpu.sync_copy(dst_hbm.at[pl.ds(base + j * K, K)], idx_v)
            pltpu.sync_copy(ones_v, acc_sh.at[idx_v], add=True)
            return carry

        lax.fori_loop(0, NCH, chunk, 0)
        plsc.subcore_barrier()
        pltpu.sync_copy(acc_sh.at[pl.ds(s * NPT, NPT)], wb_v)
        pltpu.sync_copy(wb_v, out_hbm.at[pl.ds(s * NPT, NPT), pl.ds(c * 8, 8)])

    return deg_kernel


def _make_conv(N, E, W, OC):
    TPE = E // NS
    NCH = TPE // K
    NP = _pad16x8(N)
    NPT = NP // NS

    @functools.partial(
        pl.kernel,
        out_type=jax.ShapeDtypeStruct((NP, OC), F32),
        mesh=_mesh(),
        compiler_params=pltpu.CompilerParams(use_tc_tiling_on_sc=False),
        scratch_types=[
            pltpu.VMEM((IBLK, K), jnp.int32),
            pltpu.VMEM((IBLK, K), jnp.int32),
            pltpu.VMEM((K, W), F32),
            pltpu.VMEM((K, W), F32),
            pltpu.VMEM_SHARED((NP, W), F32),
            pltpu.SemaphoreType.DMA,
            pltpu.SemaphoreType.DMA,
            pltpu.SemaphoreType.DMA,
            pltpu.SemaphoreType.DMA,
        ],
    )
    def conv_kernel(table_hbm, idx2_hbm, dst_hbm, zeros_hbm, out_hbm,
                    srcs_v, dsts_v, rows0, rows1, acc_sh, sg0, sg1, ss0, ss1):
        c = lax.axis_index("c")
        s = lax.axis_index("s")
        RPE = E // K
        pltpu.sync_copy(zeros_hbm, acc_sh.at[pl.ds(s * NPT, NPT)])
        plsc.subcore_barrier()

        rows = (rows0, rows1)
        sg = (sg0, sg1)
        ss = (ss0, ss1)

        def block(ib, carry):
            rb = c * RPE + s * NCH + ib * IBLK
            pltpu.sync_copy(idx2_hbm.at[pl.ds(rb, IBLK)], srcs_v)
            pltpu.sync_copy(dst_hbm.at[pl.ds(s * NCH + ib * IBLK, IBLK)],
                            dsts_v)
            pltpu.async_copy(table_hbm.at[srcs_v.at[0]], rows0, sg0)
            pltpu.async_copy(table_hbm.at[srcs_v.at[1]], rows1, sg1)

            def pair(t, cr):
                for p in range(2):
                    j = 2 * t + p

                    @pl.when(j >= 2)
                    def _():
                        pltpu.make_async_copy(
                            rows[p], acc_sh.at[dsts_v.at[0]], ss[p]).wait()

                    pltpu.make_async_copy(table_hbm.at[srcs_v.at[0]],
                                          rows[p], sg[p]).wait()
                    pltpu.async_copy(rows[p], acc_sh.at[dsts_v.at[j]],
                                     ss[p], add=True)

                    @pl.when(j + 2 < IBLK)
                    def _():
                        pltpu.async_copy(table_hbm.at[srcs_v.at[j + 2]],
                                         rows[p], sg[p])
                return cr

            lax.fori_loop(0, IBLK // 2, pair, 0)
            for p in range(2):
                pltpu.make_async_copy(rows[p], acc_sh.at[dsts_v.at[0]],
                                      ss[p]).wait()
            return carry

        lax.fori_loop(0, NCH // IBLK, block, 0)
        plsc.subcore_barrier()
        pltpu.sync_copy(acc_sh.at[pl.ds(s * NPT, NPT)],
                        out_hbm.at[pl.ds(s * NPT, NPT), pl.ds(c * W, W)])

    return conv_kernel


def _make_edge_gather(N, E, W):
    TPE = E // NS
    NCH = TPE // K
    BF16 = jnp.bfloat16

    @functools.partial(
        pl.kernel,
        out_type=jax.ShapeDtypeStruct((E, 2 * W), BF16),
        mesh=_mesh(),
        compiler_params=pltpu.CompilerParams(use_tc_tiling_on_sc=False),
        scratch_types=[
            pltpu.VMEM((IBLK, K), jnp.int32),
            pltpu.VMEM((IBLK, K), jnp.int32),
            pltpu.VMEM((K, W), BF16),
            pltpu.VMEM((K, W), BF16),
            pltpu.VMEM((K, W), BF16),
            pltpu.VMEM((K, W), BF16),
            pltpu.SemaphoreType.DMA,
            pltpu.SemaphoreType.DMA,
            pltpu.SemaphoreType.DMA,
            pltpu.SemaphoreType.DMA,
            pltpu.SemaphoreType.DMA,
            pltpu.SemaphoreType.DMA,
        ],
    )
    def eg_kernel(a_hbm, b_hbm, idx2_hbm, dst2_hbm, out_hbm,
                  sis_v, dis_v, ra0, rb0, ra1, rb1,
                  sa0, sb0, sa1, sb1, sw0, sw1):
        c = lax.axis_index("c")
        s = lax.axis_index("s")
        RPE = E // K
        ra = (ra0, ra1)
        rb = (rb0, rb1)
        sa = (sa0, sa1)
        sb = (sb0, sb1)
        sw = (sw0, sw1)
        base = s * TPE

        def block(ib, carry):
            rbase = c * RPE + s * NCH + ib * IBLK
            pltpu.sync_copy(idx2_hbm.at[pl.ds(rbase, IBLK)], sis_v)
            pltpu.sync_copy(dst2_hbm.at[pl.ds(rbase, IBLK)], dis_v)
            for p in range(2):
                pltpu.async_copy(a_hbm.at[sis_v.at[p]], ra[p], sa[p])
                pltpu.async_copy(b_hbm.at[dis_v.at[p]], rb[p], sb[p])

            def pair(t, cr):
                for p in range(2):
                    j = 2 * t + p
                    pltpu.make_async_copy(a_hbm.at[sis_v.at[0]],
                                          ra[p], sa[p]).wait()
                    pltpu.make_async_copy(b_hbm.at[dis_v.at[0]],
                                          rb[p], sb[p]).wait()

                    def row(r, rcr):
                        for k in range(W // 32):
                            sl = pl.ds(k * 32, 32)
                            ra[p][r, sl] = ra[p][r, sl] + rb[p][r, sl]
                        return rcr

                    lax.fori_loop(0, K, row, 0)
                    pltpu.async_copy(
                        ra[p],
                        out_hbm.at[pl.ds(base + (ib * IBLK + j) * K, K),
                                   pl.ds(c * W, W)], sw[p])

                    @pl.when(j + 2 < IBLK)
                    def _():
                        pltpu.make_async_copy(
                            ra[p],
                            out_hbm.at[pl.ds(base, K), pl.ds(c * W, W)],
                            sw[p]).wait()
                        pltpu.async_copy(a_hbm.at[sis_v.at[j + 2]],
                                         ra[p], sa[p])
                        pltpu.async_copy(b_hbm.at[dis_v.at[j + 2]],
                                         rb[p], sb[p])
                return cr

            lax.fori_loop(0, IBLK // 2, pair, 0)
            for p in range(2):
                pltpu.make_async_copy(
                    ra[p], out_hbm.at[pl.ds(base, K), pl.ds(c * W, W)],
                    sw[p]).wait()
            return carry

        lax.fori_loop(0, NCH // IBLK, block, 0)

    return eg_kernel


def _tc_idx(src_m, dst_m, N):
    R, Cm = src_m.shape

    def body(s_ref, d_ref, i2_ref, d2_ref):
        s = s_ref[...]
        d = d_ref[...]
        i2_ref[0] = s
        i2_ref[1] = s + N
        d2_ref[0] = d
        d2_ref[1] = d + N

    return pl.pallas_call(
        body,
        out_shape=[jax.ShapeDtypeStruct((2, R, Cm), jnp.int32)] * 2,
    )(src_m, dst_m)


def _tc_stats(x, batch_col):
    N, D = x.shape
    BN = 2000
    grid = (N // BN,)

    def body(x_ref, b_ref, sums_ref, cnt_ref):
        i = pl.program_id(0)
        oh = (b_ref[...] == lax.broadcasted_iota(jnp.int32, (BN, NGRP), 1)
              ).astype(F32)
        sums = lax.dot_general(oh, x_ref[...], (((0,), (0,)), ((), ())),
                               preferred_element_type=F32)
        cnt = lax.dot_general(oh, jnp.ones((BN, 128), F32),
                              (((0,), (0,)), ((), ())),
                              preferred_element_type=F32)

        @pl.when(i == 0)
        def _():
            sums_ref[...] = jnp.zeros_like(sums_ref)
            cnt_ref[...] = jnp.zeros_like(cnt_ref)

        sums_ref[...] += sums
        cnt_ref[...] += cnt

    return pl.pallas_call(
        body,
        grid=grid,
        in_specs=[pl.BlockSpec((BN, D), lambda i: (i, 0)),
                  pl.BlockSpec((BN, 1), lambda i: (i, 0))],
        out_specs=[pl.BlockSpec((NGRP, D), lambda i: (0, 0)),
                   pl.BlockSpec((NGRP, 128), lambda i: (0, 0))],
        out_shape=[jax.ShapeDtypeStruct((NGRP, D), F32),
                   jax.ShapeDtypeStruct((NGRP, 128), F32)],
    )(x, batch_col)


def _dinv_of(dg):
    deg = dg[:, 0:1] + dg[:, 8:9] + 1.0
    return lax.rsqrt(deg)


def _tc_z1(x, batch_col, degcat):
    N, D = x.shape
    BN = 2000
    grid = (N // BN,)

    def body(x_ref, b_ref, dg_ref, out_ref):
        dinv = _dinv_of(dg_ref)
        ohd = ((b_ref[...] == lax.broadcasted_iota(jnp.int32, (BN, NGRP), 1)
                ).astype(F32)) * dinv
        xd = x_ref[...] * dinv
        z = jnp.concatenate([xd, ohd, jnp.zeros((BN, NGRP), F32)], axis=1)
        out_ref[0] = z[:, :80]
        out_ref[1] = z[:, 80:]

    return pl.pallas_call(
        body,
        grid=grid,
        in_specs=[pl.BlockSpec((BN, D), lambda i: (i, 0)),
                  pl.BlockSpec((BN, 1), lambda i: (i, 0)),
                  pl.BlockSpec((BN, 128), lambda i: (i, 0))],
        out_specs=pl.BlockSpec((2, BN, 80), lambda i: (0, i, 0)),
        out_shape=jax.ShapeDtypeStruct((2, N, 80), F32),
    )(x, batch_col, degcat)


def _tc_h1(Zcat, zcat, degcat, sums_x, cnt, W1a, W1b, b1r, batch_col):
    N = zcat.shape[1]
    BN = 2000
    grid = (N // BN,)

    def body(Z_ref, z_ref, dg_ref, sx_ref, cnt_ref, wa_ref, wb_ref, b1_ref,
             bc_ref, h1_ref, sh_ref):
        i = pl.program_id(0)
        dinv = _dinv_of(dg_ref)
        means = sx_ref[...] / jnp.maximum(cnt_ref[:, 0:1], 1.0)
        Mb = jnp.dot(means, wb_ref[...], preferred_element_type=F32)
        Wzp = jnp.concatenate([wa_ref[...], Mb, jnp.zeros((NGRP, 256), F32)],
                              axis=0)
        V = (Z_ref[...][:, :160]
             + jnp.concatenate([z_ref[0], z_ref[1]], axis=1))
        pre = jnp.dot(V, Wzp, preferred_element_type=F32)
        h1 = jnp.maximum(dinv * pre + b1_ref[...], 0.0)
        h1_ref[...] = h1
        oh = (bc_ref[...] == lax.broadcasted_iota(jnp.int32, (BN, NGRP), 1)
              ).astype(F32)

        @pl.when(i == 0)
        def _():
            sh_ref[...] = jnp.zeros_like(sh_ref)

        sh_ref[...] += lax.dot_general(oh, h1, (((0,), (0,)), ((), ())),
                                       preferred_element_type=F32)

    return pl.pallas_call(
        body,
        grid=grid,
        in_specs=[pl.BlockSpec((BN, 256), lambda i: (i, 0)),
                  pl.BlockSpec((2, BN, 80), lambda i: (0, i, 0)),
                  pl.BlockSpec((BN, 128), lambda i: (i, 0)),
                  pl.BlockSpec((NGRP, 128), lambda i: (0, 0)),
                  pl.BlockSpec((NGRP, 128), lambda i: (0, 0)),
                  pl.BlockSpec((128, 256), lambda i: (0, 0)),
                  pl.BlockSpec((128, 256), lambda i: (0, 0)),
                  pl.BlockSpec((1, 256), lambda i: (0, 0)),
                  pl.BlockSpec((BN, 1), lambda i: (i, 0))],
        out_specs=[pl.BlockSpec((BN, 256), lambda i: (i, 0)),
                   pl.BlockSpec((NGRP, 256), lambda i: (0, 0))],
        out_shape=[jax.ShapeDtypeStruct((N, 256), F32),
                   jax.ShapeDtypeStruct((NGRP, 256), F32)],
    )(Zcat, zcat, degcat, sums_x, cnt, W1a, W1b, b1r, batch_col)


def _tc_prep2(h1, batch_col, degcat, sums_h1, cnt, W2a, W2b):
    N = h1.shape[0]
    BN = 2000
    grid = (N // BN,)

    def body(h_ref, b_ref, dg_ref, sh_ref, cnt_ref, wa_ref, wb_ref, out_ref):
        dinv = _dinv_of(dg_ref)
        means = sh_ref[...] / jnp.maximum(cnt_ref[:, 0:1], 1.0)
        Mb = jnp.dot(means, wb_ref[...], preferred_element_type=F32)
        oh = (b_ref[...] == lax.broadcasted_iota(jnp.int32, (BN, NGRP), 1)
              ).astype(F32)
        t2 = (jnp.dot(h_ref[...], wa_ref[...], preferred_element_type=F32)
              + jnp.dot(oh, Mb, preferred_element_type=F32))
        u2 = t2 * dinv
        out_ref[0] = u2[:, :64]
        out_ref[1] = u2[:, 64:]

    return pl.pallas_call(
        body,
        grid=grid,
        in_specs=[pl.BlockSpec((BN, 256), lambda i: (i, 0)),
                  pl.BlockSpec((BN, 1), lambda i: (i, 0)),
                  pl.BlockSpec((BN, 128), lambda i: (i, 0)),
                  pl.BlockSpec((NGRP, 256), lambda i: (0, 0)),
                  pl.BlockSpec((NGRP, 128), lambda i: (0, 0)),
                  pl.BlockSpec((256, 128), lambda i: (0, 0)),
                  pl.BlockSpec((256, 128), lambda i: (0, 0))],
        out_specs=pl.BlockSpec((2, BN, 64), lambda i: (0, i, 0)),
        out_shape=jax.ShapeDtypeStruct((2, N, 64), F32),
    )(h1, batch_col, degcat, sums_h1, cnt, W2a, W2b)


def _tc_h2p(S2cat, u2cat, degcat, b2r, WoAB):
    N = u2cat.shape[1]
    BN = 2000
    grid = (N // BN,)

    def body(s2_ref, u2_ref, dg_ref, b2_ref, wab_ref, a_ref, b_out_ref):
        dinv = _dinv_of(dg_ref)
        S2 = s2_ref[...]
        U2 = jnp.concatenate([u2_ref[0], u2_ref[1]], axis=1)
        h2 = jnp.maximum(dinv * (S2 + U2) + b2_ref[...], 0.0)
        P = jnp.dot(h2, wab_ref[...], preferred_element_type=F32)
        a_ref[0] = P[:, :64].astype(jnp.bfloat16)
        a_ref[1] = P[:, 64:128].astype(jnp.bfloat16)
        b_out_ref[0] = P[:, 128:192].astype(jnp.bfloat16)
        b_out_ref[1] = P[:, 192:].astype(jnp.bfloat16)

    return pl.pallas_call(
        body,
        grid=grid,
        in_specs=[pl.BlockSpec((BN, 128), lambda i: (i, 0)),
                  pl.BlockSpec((2, BN, 64), lambda i: (0, i, 0)),
                  pl.BlockSpec((BN, 128), lambda i: (i, 0)),
                  pl.BlockSpec((1, 128), lambda i: (0, 0)),
                  pl.BlockSpec((128, 256), lambda i: (0, 0))],
        out_specs=[pl.BlockSpec((2, BN, 64), lambda i: (0, i, 0)),
                   pl.BlockSpec((2, BN, 64), lambda i: (0, i, 0))],
        out_shape=[jax.ShapeDtypeStruct((2, N, 64), jnp.bfloat16),
                   jax.ShapeDtypeStruct((2, N, 64), jnp.bfloat16)],
    )(S2cat, u2cat, degcat, b2r, WoAB)


def _tc_final(g, edge_attr, We, ber, WoC, bor, Wf, bfr):
    E = edge_attr.shape[0]
    BE = 6400
    grid = (E // BE,)

    def body(g_ref, ea_ref, we_ref, be_ref, wc_ref, bo_ref, wf_ref, bf_ref,
             o_ref):
        e = jnp.maximum(jnp.dot(ea_ref[...], we_ref[...],
                                preferred_element_type=F32) + be_ref[...], 0.0)
        Cc = jnp.dot(e, wc_ref[...], preferred_element_type=F32)
        gf = g_ref[...].astype(F32)
        f = jnp.maximum(gf + Cc + bo_ref[...], 0.0)
        o4 = jnp.dot(f, wf_ref[...], preferred_element_type=F32) + bf_ref[...]
        o_ref[...] = o4.T

    return pl.pallas_call(
        body,
        grid=grid,
        in_specs=[pl.BlockSpec((BE, 128), lambda i: (i, 0)),
                  pl.BlockSpec((BE, 16), lambda i: (i, 0)),
                  pl.BlockSpec((16, 64), lambda i: (0, 0)),
                  pl.BlockSpec((1, 64), lambda i: (0, 0)),
                  pl.BlockSpec((64, 128), lambda i: (0, 0)),
                  pl.BlockSpec((1, 128), lambda i: (0, 0)),
                  pl.BlockSpec((128, 4), lambda i: (0, 0)),
                  pl.BlockSpec((1, 4), lambda i: (0, 0))],
        out_specs=pl.BlockSpec((4, BE), lambda i: (0, i)),
        out_shape=jax.ShapeDtypeStruct((4, E), F32),
    )(g, edge_attr, We, ber, WoC, bor, Wf, bfr)


def kernel(x, edge_index, edge_attr, batch, W1, b1, W2, b2, We, be, Wo, bo,
           Wf, bf):
    N, D = x.shape
    E = edge_index.shape[1]
    src, dst = edge_index[0], edge_index[1]
    batch_col = batch.reshape(N, 1)

    idx2_m, dst2_m = _tc_idx(src.reshape(E // 128, 128),
                             dst.reshape(E // 128, 128), N)
    idx2 = idx2_m.reshape(2 * E // K, K)
    dst2 = dst2_m.reshape(2 * E // K, K)
    dst_rows = dst.reshape(E // K, K)
    NPT = _pad16x8(N) // NS

    sums_x, cnt = _tc_stats(x, batch_col)
    degcat = _make_deg(N, E)(dst, jnp.zeros((NPT, 8), F32),
                             jnp.ones((K, 8), F32))

    zcat = _tc_z1(x, batch_col, degcat)
    Zcat = _make_conv(N, E, 80, 256)(zcat.reshape(2 * N, 80), idx2,
                                     dst_rows, jnp.zeros((NPT, 80), F32))
    h1, sums_h1 = _tc_h1(Zcat, zcat, degcat, sums_x, cnt, W1[:D], W1[D:],
                         b1.reshape(1, -1), batch_col)

    u2cat = _tc_prep2(h1, batch_col, degcat, sums_h1, cnt, W2[:256], W2[256:])
    S2cat = _make_conv(N, E, 64, 128)(u2cat.reshape(2 * N, 64), idx2,
                                      dst_rows, jnp.zeros((NPT, 64), F32))
    WoAB = jnp.concatenate([Wo[:128], Wo[128:256]], axis=1)
    Acat, Bcat = _tc_h2p(S2cat, u2cat, degcat, b2.reshape(1, -1), WoAB)

    g = _make_edge_gather(N, E, 64)(Acat.reshape(2 * N, 64),
                                    Bcat.reshape(2 * N, 64), idx2, dst2)
    out_t = _tc_final(g, edge_attr, We, be.reshape(1, -1), Wo[256:],
                      bo.reshape(1, -1), Wf, bf.reshape(1, -1))
    return out_t.T


# speedup vs baseline: 1.2900x; 1.2900x over previous
import functools

import jax
import jax.numpy as jnp
import numpy as np
from jax import lax
from jax.experimental import pallas as pl
from jax.experimental.pallas import tpu as pltpu
from jax.experimental.pallas import tpu_sc as plsc

F32 = jnp.float32
NC, NS = 2, 16
K = 80
IBLK = 50
NGRP = 16


def _mesh():
    return plsc.VectorSubcoreMesh(core_axis_name="c", subcore_axis_name="s",
                                  num_cores=NC, num_subcores=NS)


def _pad16x8(n):
    return ((n + NS * 8 - 1) // (NS * 8)) * (NS * 8)


def _make_deg(N, E):
    TPE = E // (NC * NS)
    NCH = TPE // K
    NP = _pad16x8(N)
    NPT = NP // NS

    @functools.partial(
        pl.kernel,
        out_type=jax.ShapeDtypeStruct((NP, 128), F32),
        mesh=_mesh(),
        compiler_params=pltpu.CompilerParams(use_tc_tiling_on_sc=False),
        scratch_types=[
            pltpu.VMEM((K,), jnp.int32),
            pltpu.VMEM((K, 8), F32),
            pltpu.VMEM((NPT, 8), F32),
            pltpu.VMEM_SHARED((NP, 8), F32),
        ],
    )
    def deg_kernel(dst_hbm, zeros_hbm, ones_hbm, out_hbm,
                   idx_v, ones_v, wb_v, acc_sh):
        c = lax.axis_index("c")
        s = lax.axis_index("s")
        pltpu.sync_copy(ones_hbm, ones_v)
        pltpu.sync_copy(zeros_hbm, acc_sh.at[pl.ds(s * NPT, NPT)])
        plsc.subcore_barrier()
        base = (c * NS + s) * TPE

        def chunk(j, carry):
            pltpu.sync_copy(dst_hbm.at[pl.ds(base + j * K, K)], idx_v)
            pltpu.sync_copy(ones_v, acc_sh.at[idx_v], add=True)
            return carry

        lax.fori_loop(0, NCH, chunk, 0)
        plsc.subcore_barrier()
        pltpu.sync_copy(acc_sh.at[pl.ds(s * NPT, NPT)], wb_v)
        pltpu.sync_copy(wb_v, out_hbm.at[pl.ds(s * NPT, NPT), pl.ds(c * 8, 8)])

    return deg_kernel


def _make_conv(N, E, W, slab):
    TPE = E // NS
    NCH = TPE // K
    NP = _pad16x8(N)
    NPT = NP // NS

    @functools.partial(
        pl.kernel,
        out_type=(jax.ShapeDtypeStruct((NC, NP, 128), F32) if slab
                  else jax.ShapeDtypeStruct((NP, 128), F32)),
        mesh=_mesh(),
        compiler_params=pltpu.CompilerParams(use_tc_tiling_on_sc=False),
        scratch_types=[
            pltpu.VMEM((IBLK, K), jnp.int32),
            pltpu.VMEM((IBLK, K), jnp.int32),
            pltpu.VMEM((K, W), F32),
            pltpu.VMEM((K, W), F32),
            pltpu.VMEM_SHARED((NP, W), F32),
            pltpu.SemaphoreType.DMA,
            pltpu.SemaphoreType.DMA,
            pltpu.SemaphoreType.DMA,
            pltpu.SemaphoreType.DMA,
        ],
    )
    def conv_kernel(table_hbm, idx2_hbm, dst_hbm, zeros_hbm, out_hbm,
                    srcs_v, dsts_v, rows0, rows1, acc_sh, sg0, sg1, ss0, ss1):
        c = lax.axis_index("c")
        s = lax.axis_index("s")
        RPE = E // K
        pltpu.sync_copy(zeros_hbm, acc_sh.at[pl.ds(s * NPT, NPT)])
        plsc.subcore_barrier()

        rows = (rows0, rows1)
        sg = (sg0, sg1)
        ss = (ss0, ss1)

        def block(ib, carry):
            rb = c * RPE + s * NCH + ib * IBLK
            pltpu.sync_copy(idx2_hbm.at[pl.ds(rb, IBLK)], srcs_v)
            pltpu.sync_copy(dst_hbm.at[pl.ds(s * NCH + ib * IBLK, IBLK)],
                            dsts_v)
            pltpu.async_copy(table_hbm.at[srcs_v.at[0]], rows0, sg0)
            pltpu.async_copy(table_hbm.at[srcs_v.at[1]], rows1, sg1)

            def pair(t, cr):
                for p in range(2):
                    j = 2 * t + p

                    @pl.when(j >= 2)
                    def _():
                        pltpu.make_async_copy(
                            rows[p], acc_sh.at[dsts_v.at[0]], ss[p]).wait()

                    pltpu.make_async_copy(table_hbm.at[srcs_v.at[0]],
                                          rows[p], sg[p]).wait()
                    pltpu.async_copy(rows[p], acc_sh.at[dsts_v.at[j]],
                                     ss[p], add=True)

                    @pl.when(j + 2 < IBLK)
                    def _():
                        pltpu.async_copy(table_hbm.at[srcs_v.at[j + 2]],
                                         rows[p], sg[p])
                return cr

            lax.fori_loop(0, IBLK // 2, pair, 0)
            for p in range(2):
                pltpu.make_async_copy(rows[p], acc_sh.at[dsts_v.at[0]],
                                      ss[p]).wait()
            return carry

        lax.fori_loop(0, NCH // IBLK, block, 0)
        plsc.subcore_barrier()
        if slab:
            pltpu.sync_copy(acc_sh.at[pl.ds(s * NPT, NPT)],
                            out_hbm.at[c, pl.ds(s * NPT, NPT), pl.ds(0, W)])
        else:
            pltpu.sync_copy(acc_sh.at[pl.ds(s * NPT, NPT)],
                            out_hbm.at[pl.ds(s * NPT, NPT), pl.ds(c * W, W)])

    return conv_kernel


def _make_edge_gather(N, E, W):
    TPE = E // NS
    NCH = TPE // K
    BF16 = jnp.bfloat16

    @functools.partial(
        pl.kernel,
        out_type=jax.ShapeDtypeStruct((E, 2 * W), F32),
        mesh=_mesh(),
        compiler_params=pltpu.CompilerParams(use_tc_tiling_on_sc=False,
                                             needs_layout_passes=False),
        scratch_types=[
            pltpu.VMEM((IBLK, K), jnp.int32),
            pltpu.VMEM((IBLK, K), jnp.int32),
            pltpu.VMEM((K, W), BF16),
            pltpu.VMEM((K, W), BF16),
            pltpu.VMEM((K, W), BF16),
            pltpu.VMEM((K, W), BF16),
            pltpu.VMEM((K, W), F32),
            pltpu.VMEM((K, W), F32),
            pltpu.SemaphoreType.DMA,
            pltpu.SemaphoreType.DMA,
            pltpu.SemaphoreType.DMA,
            pltpu.SemaphoreType.DMA,
            pltpu.SemaphoreType.DMA,
            pltpu.SemaphoreType.DMA,
        ],
    )
    def eg_kernel(a_hbm, b_hbm, idx2_hbm, dst2_hbm, out_hbm,
                  sis_v, dis_v, ra0, rb0, ra1, rb1, wo0, wo1,
                  sa0, sb0, sa1, sb1, sw0, sw1):
        c = lax.axis_index("c")
        s = lax.axis_index("s")
        RPE = E // K
        ra = (ra0, ra1)
        rb = (rb0, rb1)
        wo = (wo0, wo1)
        sa = (sa0, sa1)
        sb = (sb0, sb1)
        sw = (sw0, sw1)
        base = s * TPE

        def block(ib, carry):
            rbase = c * RPE + s * NCH + ib * IBLK
            pltpu.sync_copy(idx2_hbm.at[pl.ds(rbase, IBLK)], sis_v)
            pltpu.sync_copy(dst2_hbm.at[pl.ds(rbase, IBLK)], dis_v)
            for p in range(2):
                pltpu.async_copy(a_hbm.at[sis_v.at[p]], ra[p], sa[p])
                pltpu.async_copy(b_hbm.at[dis_v.at[p]], rb[p], sb[p])

            def pair(t, cr):
                for p in range(2):
                    j = 2 * t + p
                    pltpu.make_async_copy(a_hbm.at[sis_v.at[0]],
                                          ra[p], sa[p]).wait()
                    pltpu.make_async_copy(b_hbm.at[dis_v.at[0]],
                                          rb[p], sb[p]).wait()

                    @pl.when(j >= 2)
                    def _():
                        pltpu.make_async_copy(
                            wo[p],
                            out_hbm.at[pl.ds(base, K), pl.ds(c * W, W)],
                            sw[p]).wait()

                    def row(r, rcr):
                        for k in range(W // 32):
                            sl = pl.ds(k * 32, 32)
                            ssum = ra[p][r, sl] + rb[p][r, sl]
                            lo, hi = plsc.unpack(
                                ssum, format=plsc.PackFormat.INTERLEAVED)
                            wo[p][r, pl.ds(k * 32, 16)] = lo
                            wo[p][r, pl.ds(k * 32 + 16, 16)] = hi
                        return rcr

                    lax.fori_loop(0, K, row, 0)
                    pltpu.async_copy(
                        wo[p],
                        out_hbm.at[pl.ds(base + (ib * IBLK + j) * K, K),
                                   pl.ds(c * W, W)], sw[p])

                    @pl.when(j + 2 < IBLK)
                    def _():
                        pltpu.async_copy(a_hbm.at[sis_v.at[j + 2]],
                                         ra[p], sa[p])
                        pltpu.async_copy(b_hbm.at[dis_v.at[j + 2]],
                                         rb[p], sb[p])
                return cr

            lax.fori_loop(0, IBLK // 2, pair, 0)
            for p in range(2):
                pltpu.make_async_copy(
                    wo[p], out_hbm.at[pl.ds(base, K), pl.ds(c * W, W)],
                    sw[p]).wait()
            return carry

        lax.fori_loop(0, NCH // IBLK, block, 0)

    return eg_kernel


def _tc_idx(src_m, dst_m, N):
    R, Cm = src_m.shape

    def body(s_ref, d_ref, i2_ref, d2_ref):
        s = s_ref[...]
        d = d_ref[...]
        i2_ref[0] = s
        i2_ref[1] = s + N
        d2_ref[0] = d
        d2_ref[1] = d + N

    return pl.pallas_call(
        body,
        out_shape=[jax.ShapeDtypeStruct((2, R, Cm), jnp.int32)] * 2,
    )(src_m, dst_m)


def _tc_stats(x, batch_col):
    N, D = x.shape
    BN = 2000
    grid = (N // BN,)

    def body(x_ref, b_ref, sums_ref, cnt_ref):
        i = pl.program_id(0)
        oh = (b_ref[...] == lax.broadcasted_iota(jnp.int32, (BN, NGRP), 1)
              ).astype(F32)
        sums = lax.dot_general(oh, x_ref[...], (((0,), (0,)), ((), ())),
                               preferred_element_type=F32)
        cnt = lax.dot_general(oh, jnp.ones((BN, 128), F32),
                              (((0,), (0,)), ((), ())),
                              preferred_element_type=F32)

        @pl.when(i == 0)
        def _():
            sums_ref[...] = jnp.zeros_like(sums_ref)
            cnt_ref[...] = jnp.zeros_like(cnt_ref)

        sums_ref[...] += sums
        cnt_ref[...] += cnt

    return pl.pallas_call(
        body,
        grid=grid,
        in_specs=[pl.BlockSpec((BN, D), lambda i: (i, 0)),
                  pl.BlockSpec((BN, 1), lambda i: (i, 0))],
        out_specs=[pl.BlockSpec((NGRP, D), lambda i: (0, 0)),
                   pl.BlockSpec((NGRP, 128), lambda i: (0, 0))],
        out_shape=[jax.ShapeDtypeStruct((NGRP, D), F32),
                   jax.ShapeDtypeStruct((NGRP, 128), F32)],
    )(x, batch_col)


def _dinv_of(dg):
    deg = dg[:, 0:1] + dg[:, 8:9] + 1.0
    return lax.rsqrt(deg)


def _tc_z1(x, batch_col, degcat):
    N, D = x.shape
    BN = 2000
    grid = (N // BN,)

    def body(x_ref, b_ref, dg_ref, out_ref):
        dinv = _dinv_of(dg_ref)
        ohd = ((b_ref[...] == lax.broadcasted_iota(jnp.int32, (BN, NGRP), 1)
                ).astype(F32)) * dinv
        xd = x_ref[...] * dinv
        z = jnp.concatenate([xd, ohd, jnp.zeros((BN, NGRP), F32)], axis=1)
        out_ref[0] = z[:, :80]
        out_ref[1] = z[:, 80:]

    return pl.pallas_call(
        body,
        grid=grid,
        in_specs=[pl.BlockSpec((BN, D), lambda i: (i, 0)),
                  pl.BlockSpec((BN, 1), lambda i: (i, 0)),
                  pl.BlockSpec((BN, 128), lambda i: (i, 0))],
        out_specs=pl.BlockSpec((2, BN, 80), lambda i: (0, i, 0)),
        out_shape=jax.ShapeDtypeStruct((2, N, 80), F32),
    )(x, batch_col, degcat)


def _tc_h1(Zcat, zcat, degcat, sums_x, cnt, W1a, W1b, b1r, batch_col):
    N = zcat.shape[1]
    BN = 2000
    grid = (N // BN,)

    def body(Z_ref, z_ref, dg_ref, sx_ref, cnt_ref, wa_ref, wb_ref, b1_ref,
             bc_ref, h1_ref, sh_ref):
        i = pl.program_id(0)
        dinv = _dinv_of(dg_ref)
        means = sx_ref[...] / jnp.maximum(cnt_ref[:, 0:1], 1.0)
        Mb = jnp.dot(means, wb_ref[...], preferred_element_type=F32)
        Wzp = jnp.concatenate([wa_ref[...], Mb, jnp.zeros((NGRP, 256), F32)],
                              axis=0)
        V = (jnp.concatenate([Z_ref[0][:, :80], Z_ref[1][:, :80]], axis=1)
             + jnp.concatenate([z_ref[0], z_ref[1]], axis=1))
        pre = jnp.dot(V, Wzp, preferred_element_type=F32)
        h1 = jnp.maximum(dinv * pre + b1_ref[...], 0.0)
        h1_ref[...] = h1
        oh = (bc_ref[...] == lax.broadcasted_iota(jnp.int32, (BN, NGRP), 1)
              ).astype(F32)

        @pl.when(i == 0)
        def _():
            sh_ref[...] = jnp.zeros_like(sh_ref)

        sh_ref[...] += lax.dot_general(oh, h1, (((0,), (0,)), ((), ())),
                                       preferred_element_type=F32)

    return pl.pallas_call(
        body,
        grid=grid,
        in_specs=[pl.BlockSpec((2, BN, 128), lambda i: (0, i, 0)),
                  pl.BlockSpec((2, BN, 80), lambda i: (0, i, 0)),
                  pl.BlockSpec((BN, 128), lambda i: (i, 0)),
                  pl.BlockSpec((NGRP, 128), lambda i: (0, 0)),
                  pl.BlockSpec((NGRP, 128), lambda i: (0, 0)),
                  pl.BlockSpec((128, 256), lambda i: (0, 0)),
                  pl.BlockSpec((128, 256), lambda i: (0, 0)),
                  pl.BlockSpec((1, 256), lambda i: (0, 0)),
                  pl.BlockSpec((BN, 1), lambda i: (i, 0))],
        out_specs=[pl.BlockSpec((BN, 256), lambda i: (i, 0)),
                   pl.BlockSpec((NGRP, 256), lambda i: (0, 0))],
        out_shape=[jax.ShapeDtypeStruct((N, 256), F32),
                   jax.ShapeDtypeStruct((NGRP, 256), F32)],
    )(Zcat, zcat, degcat, sums_x, cnt, W1a, W1b, b1r, batch_col)


def _tc_prep2(h1, batch_col, degcat, sums_h1, cnt, W2a, W2b):
    N = h1.shape[0]
    BN = 2000
    grid = (N // BN,)

    def body(h_ref, b_ref, dg_ref, sh_ref, cnt_ref, wa_ref, wb_ref, out_ref):
        dinv = _dinv_of(dg_ref)
        means = sh_ref[...] / jnp.maximum(cnt_ref[:, 0:1], 1.0)
        Mb = jnp.dot(means, wb_ref[...], preferred_element_type=F32)
        oh = (b_ref[...] == lax.broadcasted_iota(jnp.int32, (BN, NGRP), 1)
              ).astype(F32)
        t2 = (jnp.dot(h_ref[...], wa_ref[...], preferred_element_type=F32)
              + jnp.dot(oh, Mb, preferred_element_type=F32))
        u2 = t2 * dinv
        out_ref[0] = u2[:, :64]
        out_ref[1] = u2[:, 64:]

    return pl.pallas_call(
        body,
        grid=grid,
        in_specs=[pl.BlockSpec((BN, 256), lambda i: (i, 0)),
                  pl.BlockSpec((BN, 1), lambda i: (i, 0)),
                  pl.BlockSpec((BN, 128), lambda i: (i, 0)),
                  pl.BlockSpec((NGRP, 256), lambda i: (0, 0)),
                  pl.BlockSpec((NGRP, 128), lambda i: (0, 0)),
                  pl.BlockSpec((256, 128), lambda i: (0, 0)),
                  pl.BlockSpec((256, 128), lambda i: (0, 0))],
        out_specs=pl.BlockSpec((2, BN, 64), lambda i: (0, i, 0)),
        out_shape=jax.ShapeDtypeStruct((2, N, 64), F32),
    )(h1, batch_col, degcat, sums_h1, cnt, W2a, W2b)


def _tc_h2p(S2cat, u2cat, degcat, b2r, WoAB):
    N = u2cat.shape[1]
    BN = 2000
    grid = (N // BN,)

    def body(s2_ref, u2_ref, dg_ref, b2_ref, wab_ref, a_ref, b_out_ref):
        dinv = _dinv_of(dg_ref)
        S2 = s2_ref[...]
        U2 = jnp.concatenate([u2_ref[0], u2_ref[1]], axis=1)
        h2 = jnp.maximum(dinv * (S2 + U2) + b2_ref[...], 0.0)
        P = jnp.dot(h2, wab_ref[...], preferred_element_type=F32)
        a_ref[0] = P[:, :64].astype(jnp.bfloat16)
        a_ref[1] = P[:, 64:128].astype(jnp.bfloat16)
        b_out_ref[0] = P[:, 128:192].astype(jnp.bfloat16)
        b_out_ref[1] = P[:, 192:].astype(jnp.bfloat16)

    return pl.pallas_call(
        body,
        grid=grid,
        in_specs=[pl.BlockSpec((BN, 128), lambda i: (i, 0)),
                  pl.BlockSpec((2, BN, 64), lambda i: (0, i, 0)),
                  pl.BlockSpec((BN, 128), lambda i: (i, 0)),
                  pl.BlockSpec((1, 128), lambda i: (0, 0)),
                  pl.BlockSpec((128, 256), lambda i: (0, 0))],
        out_specs=[pl.BlockSpec((2, BN, 64), lambda i: (0, i, 0)),
                   pl.BlockSpec((2, BN, 64), lambda i: (0, i, 0))],
        out_shape=[jax.ShapeDtypeStruct((2, N, 64), jnp.bfloat16),
                   jax.ShapeDtypeStruct((2, N, 64), jnp.bfloat16)],
    )(S2cat, u2cat, degcat, b2r, WoAB)


def _tc_final(g, edge_attr, We, ber, WoC, bor, Wf, bfr):
    E = edge_attr.shape[0]
    BE = 6400
    grid = (E // BE,)

    def body(g_ref, ea_ref, we_ref, be_ref, wc_ref, bo_ref, wf_ref, bf_ref,
             o_ref):
        e = jnp.maximum(jnp.dot(ea_ref[...], we_ref[...],
                                preferred_element_type=F32) + be_ref[...], 0.0)
        Cc = jnp.dot(e, wc_ref[...], preferred_element_type=F32)
        gf = g_ref[...]
        f = jnp.maximum(gf + Cc + bo_ref[...], 0.0)
        o4 = jnp.dot(f, wf_ref[...], preferred_element_type=F32) + bf_ref[...]
        o_ref[...] = o4.T

    return pl.pallas_call(
        body,
        grid=grid,
        in_specs=[pl.BlockSpec((BE, 128), lambda i: (i, 0)),
                  pl.BlockSpec((BE, 16), lambda i: (i, 0)),
                  pl.BlockSpec((16, 64), lambda i: (0, 0)),
                  pl.BlockSpec((1, 64), lambda i: (0, 0)),
                  pl.BlockSpec((64, 128), lambda i: (0, 0)),
                  pl.BlockSpec((1, 128), lambda i: (0, 0)),
                  pl.BlockSpec((128, 4), lambda i: (0, 0)),
                  pl.BlockSpec((1, 4), lambda i: (0, 0))],
        out_specs=pl.BlockSpec((4, BE), lambda i: (0, i)),
        out_shape=jax.ShapeDtypeStruct((4, E), F32),
    )(g, edge_attr, We, ber, WoC, bor, Wf, bfr)


def kernel(x, edge_index, edge_attr, batch, W1, b1, W2, b2, We, be, Wo, bo,
           Wf, bf):
    N, D = x.shape
    E = edge_index.shape[1]
    src, dst = edge_index[0], edge_index[1]
    batch_col = batch.reshape(N, 1)

    idx2_m, dst2_m = _tc_idx(src.reshape(E // 128, 128),
                             dst.reshape(E // 128, 128), N)
    idx2 = idx2_m.reshape(2 * E // K, K)
    dst2 = dst2_m.reshape(2 * E // K, K)
    dst_rows = dst.reshape(E // K, K)
    NPT = _pad16x8(N) // NS

    sums_x, cnt = _tc_stats(x, batch_col)
    degcat = _make_deg(N, E)(dst, jnp.zeros((NPT, 8), F32),
                             jnp.ones((K, 8), F32))

    zcat = _tc_z1(x, batch_col, degcat)
    Zcat = _make_conv(N, E, 80, True)(zcat.reshape(2 * N, 80), idx2,
                                      dst_rows, jnp.zeros((NPT, 80), F32))
    h1, sums_h1 = _tc_h1(Zcat, zcat, degcat, sums_x, cnt, W1[:D], W1[D:],
                         b1.reshape(1, -1), batch_col)

    u2cat = _tc_prep2(h1, batch_col, degcat, sums_h1, cnt, W2[:256], W2[256:])
    S2cat = _make_conv(N, E, 64, False)(u2cat.reshape(2 * N, 64), idx2,
                                        dst_rows, jnp.zeros((NPT, 64), F32))
    WoAB = jnp.concatenate([Wo[:128], Wo[128:256]], axis=1)
    Acat, Bcat = _tc_h2p(S2cat, u2cat, degcat, b2.reshape(1, -1), WoAB)

    g = _make_edge_gather(N, E, 64)(Acat.reshape(2 * N, 64),
                                    Bcat.reshape(2 * N, 64), idx2, dst2)
    f_of = np.empty(128, np.int32)
    for f in range(128):
        cc, tt = f // 64, f % 64
        blk, q = tt // 32, tt % 32
        f_of[cc * 64 + blk * 32 + q // 2 + (16 if q % 2 else 0)] = f
    out_t = _tc_final(g, edge_attr, We, be.reshape(1, -1), Wo[256:][:, f_of],
                      bo[f_of].reshape(1, -1), Wf[f_of],
                      bf.reshape(1, -1))
    return out_t.T


# speedup vs baseline: 1.5647x; 1.2129x over previous
import functools

import jax
import jax.numpy as jnp
import numpy as np
from jax import lax
from jax.experimental import pallas as pl
from jax.experimental.pallas import tpu as pltpu
from jax.experimental.pallas import tpu_sc as plsc

F32 = jnp.float32
NC, NS = 2, 16
K = 80
IBLK = 50
NGRP = 16


def _mesh():
    return plsc.VectorSubcoreMesh(core_axis_name="c", subcore_axis_name="s",
                                  num_cores=NC, num_subcores=NS)


def _pad16x8(n):
    return ((n + NS * 8 - 1) // (NS * 8)) * (NS * 8)


def _make_deg(N, E):
    TPE = E // (NC * NS)
    RPT = TPE // K
    NP = _pad16x8(N)
    NPT = NP // NS

    @functools.partial(
        pl.kernel,
        out_type=jax.ShapeDtypeStruct((NP, 128), F32),
        mesh=_mesh(),
        compiler_params=pltpu.CompilerParams(use_tc_tiling_on_sc=False),
        scratch_types=[
            pltpu.VMEM((RPT, K), jnp.int32),
            pltpu.VMEM((K, 8), F32),
            pltpu.VMEM((NPT, 8), F32),
            pltpu.VMEM_SHARED((NP, 8), F32),
            pltpu.SemaphoreType.DMA,
        ],
    )
    def deg_kernel(dstr_hbm, zeros_hbm, ones_hbm, out_hbm,
                   dsts_v, ones_v, wb_v, acc_sh, ssem):
        c = lax.axis_index("c")
        s = lax.axis_index("s")
        pltpu.sync_copy(ones_hbm, ones_v)
        pltpu.sync_copy(zeros_hbm, acc_sh.at[pl.ds(s * NPT, NPT)])
        pltpu.sync_copy(dstr_hbm.at[pl.ds((c * NS + s) * RPT, RPT)], dsts_v)
        plsc.subcore_barrier()

        def chunk(j, carry):
            pltpu.sync_copy(ones_v, acc_sh.at[dsts_v.at[j]], add=True)
            return carry

        lax.fori_loop(0, RPT, chunk, 0)
        plsc.subcore_barrier()
        pltpu.sync_copy(acc_sh.at[pl.ds(s * NPT, NPT)], wb_v)
        pltpu.sync_copy(wb_v, out_hbm.at[pl.ds(s * NPT, NPT), pl.ds(c * 8, 8)])

    return deg_kernel


def _make_conv(N, E, W, slab):
    TPE = E // NS
    NCH = TPE // K
    NP = _pad16x8(N)
    NPT = NP // NS
    NB = 5

    @functools.partial(
        pl.kernel,
        out_type=(jax.ShapeDtypeStruct((NC, NP, 128), F32) if slab
                  else jax.ShapeDtypeStruct((NP, 128), F32)),
        mesh=_mesh(),
        compiler_params=pltpu.CompilerParams(use_tc_tiling_on_sc=False),
        scratch_types=[
            pltpu.VMEM((IBLK, K), jnp.int32),
            pltpu.VMEM((IBLK, K), jnp.int32),
        ] + [pltpu.VMEM((K, W), F32)] * NB + [
            pltpu.VMEM_SHARED((NP, W), F32),
        ] + [pltpu.SemaphoreType.DMA] * (2 * NB),
    )
    def conv_kernel(table_hbm, idx2_hbm, dst_hbm, zeros_hbm, out_hbm,
                    srcs_v, dsts_v, *rest):
        rows = rest[:NB]
        acc_sh = rest[NB]
        sg = rest[NB + 1:NB + 1 + NB]
        ss = rest[NB + 1 + NB:]
        c = lax.axis_index("c")
        s = lax.axis_index("s")
        RPE = E // K
        pltpu.sync_copy(zeros_hbm, acc_sh.at[pl.ds(s * NPT, NPT)])
        plsc.subcore_barrier()

        def block(ib, carry):
            rb = c * RPE + s * NCH + ib * IBLK
            pltpu.sync_copy(idx2_hbm.at[pl.ds(rb, IBLK)], srcs_v)
            pltpu.sync_copy(dst_hbm.at[pl.ds(s * NCH + ib * IBLK, IBLK)],
                            dsts_v)
            for p in range(NB):
                pltpu.async_copy(table_hbm.at[srcs_v.at[p]], rows[p], sg[p])

            def quad(t, cr):
                for p in range(NB):
                    j = NB * t + p
                    pv = (p - 1) % NB
                    jr = j - 1 + NB

                    @pl.when((j >= 1) & (jr < IBLK))
                    def _():
                        pltpu.make_async_copy(
                            rows[pv], acc_sh.at[dsts_v.at[0]], ss[pv]).wait()
                        pltpu.async_copy(table_hbm.at[srcs_v.at[jr]],
                                         rows[pv], sg[pv])

                    pltpu.make_async_copy(table_hbm.at[srcs_v.at[0]],
                                          rows[p], sg[p]).wait()
                    pltpu.async_copy(rows[p], acc_sh.at[dsts_v.at[j]],
                                     ss[p], add=True)
                return cr

            lax.fori_loop(0, IBLK // NB, quad, 0)
            for p in range(NB):
                pltpu.make_async_copy(rows[p], acc_sh.at[dsts_v.at[0]],
                                      ss[p]).wait()
            return carry

        lax.fori_loop(0, NCH // IBLK, block, 0)
        plsc.subcore_barrier()
        if slab:
            pltpu.sync_copy(acc_sh.at[pl.ds(s * NPT, NPT)],
                            out_hbm.at[c, pl.ds(s * NPT, NPT), pl.ds(0, W)])
        else:
            pltpu.sync_copy(acc_sh.at[pl.ds(s * NPT, NPT)],
                            out_hbm.at[pl.ds(s * NPT, NPT), pl.ds(c * W, W)])

    return conv_kernel


def _make_edge_gather(N, E, W):
    TPE = E // NS
    NCH = TPE // K

    @functools.partial(
        pl.kernel,
        out_type=jax.ShapeDtypeStruct((E, 2 * W), F32),
        mesh=_mesh(),
        compiler_params=pltpu.CompilerParams(use_tc_tiling_on_sc=False),
        scratch_types=[
            pltpu.VMEM((IBLK, K), jnp.int32),
            pltpu.VMEM((IBLK, K), jnp.int32),
            pltpu.VMEM((K, W), F32),
            pltpu.VMEM((K, W), F32),
            pltpu.VMEM((K, W), F32),
            pltpu.VMEM((K, W), F32),
            pltpu.SemaphoreType.DMA,
            pltpu.SemaphoreType.DMA,
            pltpu.SemaphoreType.DMA,
            pltpu.SemaphoreType.DMA,
            pltpu.SemaphoreType.DMA,
            pltpu.SemaphoreType.DMA,
        ],
    )
    def eg_kernel(a_hbm, b_hbm, idx2_hbm, dst2_hbm, out_hbm,
                  sis_v, dis_v, ra0, rb0, ra1, rb1,
                  sa0, sb0, sa1, sb1, sw0, sw1):
        c = lax.axis_index("c")
        s = lax.axis_index("s")
        RPE = E // K
        ra = (ra0, ra1)
        rb = (rb0, rb1)
        sa = (sa0, sa1)
        sb = (sb0, sb1)
        sw = (sw0, sw1)
        base = s * TPE

        def block(ib, carry):
            rbase = c * RPE + s * NCH + ib * IBLK
            pltpu.sync_copy(idx2_hbm.at[pl.ds(rbase, IBLK)], sis_v)
            pltpu.sync_copy(dst2_hbm.at[pl.ds(rbase, IBLK)], dis_v)
            for p in range(2):
                pltpu.async_copy(a_hbm.at[sis_v.at[p]], ra[p], sa[p])
                pltpu.async_copy(b_hbm.at[dis_v.at[p]], rb[p], sb[p])

            def pair(t, cr):
                for p in range(2):
                    j = 2 * t + p
                    pltpu.make_async_copy(a_hbm.at[sis_v.at[0]],
                                          ra[p], sa[p]).wait()
                    pltpu.make_async_copy(b_hbm.at[dis_v.at[0]],
                                          rb[p], sb[p]).wait()

                    def row(r, rcr):
                        for k in range(W // 16):
                            sl = pl.ds(k * 16, 16)
                            ra[p][r, sl] = ra[p][r, sl] + rb[p][r, sl]
                        return rcr

                    lax.fori_loop(0, K, row, 0)
                    pltpu.async_copy(
                        ra[p],
                        out_hbm.at[pl.ds(base + (ib * IBLK + j) * K, K),
                                   pl.ds(c * W, W)], sw[p])

                    @pl.when(j + 2 < IBLK)
                    def _():
                        pltpu.make_async_copy(
                            ra[p],
                            out_hbm.at[pl.ds(base, K), pl.ds(c * W, W)],
                            sw[p]).wait()
                        pltpu.async_copy(a_hbm.at[sis_v.at[j + 2]],
                                         ra[p], sa[p])
                        pltpu.async_copy(b_hbm.at[dis_v.at[j + 2]],
                                         rb[p], sb[p])
                return cr

            lax.fori_loop(0, IBLK // 2, pair, 0)
            for p in range(2):
                pltpu.make_async_copy(
                    ra[p], out_hbm.at[pl.ds(base, K), pl.ds(c * W, W)],
                    sw[p]).wait()
            return carry

        lax.fori_loop(0, NCH // IBLK, block, 0)

    return eg_kernel


def _tc_idx(src_m, dst_m, N):
    R, Cm = src_m.shape

    def body(s_ref, d_ref, i2_ref, d2_ref):
        s = s_ref[...]
        d = d_ref[...]
        i2_ref[0] = s
        i2_ref[1] = s + N
        d2_ref[0] = d
        d2_ref[1] = d + N

    return pl.pallas_call(
        body,
        out_shape=[jax.ShapeDtypeStruct((2, R, Cm), jnp.int32)] * 2,
    )(src_m, dst_m)


def _tc_stats(x, batch_col):
    N, D = x.shape
    BN = 2000
    grid = (N // BN,)

    def body(x_ref, b_ref, sums_ref, cnt_ref):
        i = pl.program_id(0)
        oh = (b_ref[...] == lax.broadcasted_iota(jnp.int32, (BN, NGRP), 1)
              ).astype(F32)
        sums = lax.dot_general(oh, x_ref[...], (((0,), (0,)), ((), ())),
                               preferred_element_type=F32)
        cnt = lax.dot_general(oh, jnp.ones((BN, 128), F32),
                              (((0,), (0,)), ((), ())),
                              preferred_element_type=F32)

        @pl.when(i == 0)
        def _():
            sums_ref[...] = jnp.zeros_like(sums_ref)
            cnt_ref[...] = jnp.zeros_like(cnt_ref)

        sums_ref[...] += sums
        cnt_ref[...] += cnt

    return pl.pallas_call(
        body,
        grid=grid,
        in_specs=[pl.BlockSpec((BN, D), lambda i: (i, 0)),
                  pl.BlockSpec((BN, 1), lambda i: (i, 0))],
        out_specs=[pl.BlockSpec((NGRP, D), lambda i: (0, 0)),
                   pl.BlockSpec((NGRP, 128), lambda i: (0, 0))],
        out_shape=[jax.ShapeDtypeStruct((NGRP, D), F32),
                   jax.ShapeDtypeStruct((NGRP, 128), F32)],
    )(x, batch_col)


def _dinv_of(dg):
    deg = dg[:, 0:1] + dg[:, 8:9] + 1.0
    return lax.rsqrt(deg)


def _tc_z1(x, batch_col, degcat):
    N, D = x.shape
    BN = 2000
    grid = (N // BN,)

    def body(x_ref, b_ref, dg_ref, out_ref):
        dinv = _dinv_of(dg_ref)
        ohd = ((b_ref[...] == lax.broadcasted_iota(jnp.int32, (BN, NGRP), 1)
                ).astype(F32)) * dinv
        xd = x_ref[...] * dinv
        z = jnp.concatenate([xd, ohd, jnp.zeros((BN, NGRP), F32)], axis=1)
        out_ref[0] = z[:, :80]
        out_ref[1] = z[:, 80:]

    return pl.pallas_call(
        body,
        grid=grid,
        in_specs=[pl.BlockSpec((BN, D), lambda i: (i, 0)),
                  pl.BlockSpec((BN, 1), lambda i: (i, 0)),
                  pl.BlockSpec((BN, 128), lambda i: (i, 0))],
        out_specs=pl.BlockSpec((2, BN, 80), lambda i: (0, i, 0)),
        out_shape=jax.ShapeDtypeStruct((2, N, 80), F32),
    )(x, batch_col, degcat)


def _tc_h1(Zcat, zcat, degcat, sums_x, cnt, W1a, W1b, b1r, batch_col):
    N = zcat.shape[1]
    BN = 2000
    grid = (N // BN,)

    def body(Z_ref, z_ref, dg_ref, sx_ref, cnt_ref, wa_ref, wb_ref, b1_ref,
             bc_ref, h1_ref, sh_ref):
        i = pl.program_id(0)
        dinv = _dinv_of(dg_ref)
        means = sx_ref[...] / jnp.maximum(cnt_ref[:, 0:1], 1.0)
        Mb = jnp.dot(means, wb_ref[...], preferred_element_type=F32)
        Wzp = jnp.concatenate([wa_ref[...], Mb, jnp.zeros((NGRP, 256), F32)],
                              axis=0)
        V = (jnp.concatenate([Z_ref[0][:, :80], Z_ref[1][:, :80]], axis=1)
             + jnp.concatenate([z_ref[0], z_ref[1]], axis=1))
        pre = jnp.dot(V, Wzp, preferred_element_type=F32)
        h1 = jnp.maximum(dinv * pre + b1_ref[...], 0.0)
        h1_ref[...] = h1
        oh = (bc_ref[...] == lax.broadcasted_iota(jnp.int32, (BN, NGRP), 1)
              ).astype(F32)

        @pl.when(i == 0)
        def _():
            sh_ref[...] = jnp.zeros_like(sh_ref)

        sh_ref[...] += lax.dot_general(oh, h1, (((0,), (0,)), ((), ())),
                                       preferred_element_type=F32)

    return pl.pallas_call(
        body,
        grid=grid,
        in_specs=[pl.BlockSpec((2, BN, 128), lambda i: (0, i, 0)),
                  pl.BlockSpec((2, BN, 80), lambda i: (0, i, 0)),
                  pl.BlockSpec((BN, 128), lambda i: (i, 0)),
                  pl.BlockSpec((NGRP, 128), lambda i: (0, 0)),
                  pl.BlockSpec((NGRP, 128), lambda i: (0, 0)),
                  pl.BlockSpec((128, 256), lambda i: (0, 0)),
                  pl.BlockSpec((128, 256), lambda i: (0, 0)),
                  pl.BlockSpec((1, 256), lambda i: (0, 0)),
                  pl.BlockSpec((BN, 1), lambda i: (i, 0))],
        out_specs=[pl.BlockSpec((BN, 256), lambda i: (i, 0)),
                   pl.BlockSpec((NGRP, 256), lambda i: (0, 0))],
        out_shape=[jax.ShapeDtypeStruct((N, 256), F32),
                   jax.ShapeDtypeStruct((NGRP, 256), F32)],
    )(Zcat, zcat, degcat, sums_x, cnt, W1a, W1b, b1r, batch_col)


def _tc_prep2(h1, batch_col, degcat, sums_h1, cnt, W2a, W2b):
    N = h1.shape[0]
    BN = 2000
    grid = (N // BN,)

    def body(h_ref, b_ref, dg_ref, sh_ref, cnt_ref, wa_ref, wb_ref, out_ref):
        dinv = _dinv_of(dg_ref)
        means = sh_ref[...] / jnp.maximum(cnt_ref[:, 0:1], 1.0)
        Mb = jnp.dot(means, wb_ref[...], preferred_element_type=F32)
        oh = (b_ref[...] == lax.broadcasted_iota(jnp.int32, (BN, NGRP), 1)
              ).astype(F32)
        t2 = (jnp.dot(h_ref[...], wa_ref[...], preferred_element_type=F32)
              + jnp.dot(oh, Mb, preferred_element_type=F32))
        u2 = t2 * dinv
        out_ref[0] = u2[:, :64]
        out_ref[1] = u2[:, 64:]

    return pl.pallas_call(
        body,
        grid=grid,
        in_specs=[pl.BlockSpec((BN, 256), lambda i: (i, 0)),
                  pl.BlockSpec((BN, 1), lambda i: (i, 0)),
                  pl.BlockSpec((BN, 128), lambda i: (i, 0)),
                  pl.BlockSpec((NGRP, 256), lambda i: (0, 0)),
                  pl.BlockSpec((NGRP, 128), lambda i: (0, 0)),
                  pl.BlockSpec((256, 128), lambda i: (0, 0)),
                  pl.BlockSpec((256, 128), lambda i: (0, 0))],
        out_specs=pl.BlockSpec((2, BN, 64), lambda i: (0, i, 0)),
        out_shape=jax.ShapeDtypeStruct((2, N, 64), F32),
    )(h1, batch_col, degcat, sums_h1, cnt, W2a, W2b)


def _tc_h2p(S2cat, u2cat, degcat, b2r, WoAB):
    N = u2cat.shape[1]
    BN = 2000
    grid = (N // BN,)

    def body(s2_ref, u2_ref, dg_ref, b2_ref, wab_ref, a_ref, b_out_ref):
        dinv = _dinv_of(dg_ref)
        S2 = s2_ref[...]
        U2 = jnp.concatenate([u2_ref[0], u2_ref[1]], axis=1)
        h2 = jnp.maximum(dinv * (S2 + U2) + b2_ref[...], 0.0)
        P = jnp.dot(h2, wab_ref[...], preferred_element_type=F32)
        a_ref[0] = P[:, :64]
        a_ref[1] = P[:, 64:128]
        b_out_ref[0] = P[:, 128:192]
        b_out_ref[1] = P[:, 192:]

    return pl.pallas_call(
        body,
        grid=grid,
        in_specs=[pl.BlockSpec((BN, 128), lambda i: (i, 0)),
                  pl.BlockSpec((2, BN, 64), lambda i: (0, i, 0)),
                  pl.BlockSpec((BN, 128), lambda i: (i, 0)),
                  pl.BlockSpec((1, 128), lambda i: (0, 0)),
                  pl.BlockSpec((128, 256), lambda i: (0, 0))],
        out_specs=[pl.BlockSpec((2, BN, 64), lambda i: (0, i, 0)),
                   pl.BlockSpec((2, BN, 64), lambda i: (0, i, 0))],
        out_shape=[jax.ShapeDtypeStruct((2, N, 64), F32),
                   jax.ShapeDtypeStruct((2, N, 64), F32)],
    )(S2cat, u2cat, degcat, b2r, WoAB)


def _tc_final(g, edge_attr, We, ber, WoC, bor, Wf, bfr):
    E = edge_attr.shape[0]
    BE = 6400
    grid = (E // BE,)

    def body(g_ref, ea_ref, we_ref, be_ref, wc_ref, bo_ref, wf_ref, bf_ref,
             o_ref):
        e = jnp.maximum(jnp.dot(ea_ref[...], we_ref[...],
                                preferred_element_type=F32) + be_ref[...], 0.0)
        Cc = jnp.dot(e, wc_ref[...], preferred_element_type=F32)
        gf = g_ref[...]
        f = jnp.maximum(gf + Cc + bo_ref[...], 0.0)
        o4 = jnp.dot(f, wf_ref[...], preferred_element_type=F32) + bf_ref[...]
        o_ref[...] = o4.T

    return pl.pallas_call(
        body,
        grid=grid,
        in_specs=[pl.BlockSpec((BE, 128), lambda i: (i, 0)),
                  pl.BlockSpec((BE, 16), lambda i: (i, 0)),
                  pl.BlockSpec((16, 64), lambda i: (0, 0)),
                  pl.BlockSpec((1, 64), lambda i: (0, 0)),
                  pl.BlockSpec((64, 128), lambda i: (0, 0)),
                  pl.BlockSpec((1, 128), lambda i: (0, 0)),
                  pl.BlockSpec((128, 4), lambda i: (0, 0)),
                  pl.BlockSpec((1, 4), lambda i: (0, 0))],
        out_specs=pl.BlockSpec((4, BE), lambda i: (0, i)),
        out_shape=jax.ShapeDtypeStruct((4, E), F32),
    )(g, edge_attr, We, ber, WoC, bor, Wf, bfr)


def kernel(x, edge_index, edge_attr, batch, W1, b1, W2, b2, We, be, Wo, bo,
           Wf, bf):
    N, D = x.shape
    E = edge_index.shape[1]
    src, dst = edge_index[0], edge_index[1]
    batch_col = batch.reshape(N, 1)

    idx2_m, dst2_m = _tc_idx(src.reshape(E // 128, 128),
                             dst.reshape(E // 128, 128), N)
    idx2 = idx2_m.reshape(2 * E // K, K)
    dst2 = dst2_m.reshape(2 * E // K, K)
    dst_rows = dst.reshape(E // K, K)
    NPT = _pad16x8(N) // NS

    sums_x, cnt = _tc_stats(x, batch_col)
    degcat = _make_deg(N, E)(dst_rows, jnp.zeros((NPT, 8), F32),
                             jnp.ones((K, 8), F32))

    zcat = _tc_z1(x, batch_col, degcat)
    Zcat = _make_conv(N, E, 80, True)(zcat.reshape(2 * N, 80), idx2,
                                      dst_rows, jnp.zeros((NPT, 80), F32))
    h1, sums_h1 = _tc_h1(Zcat, zcat, degcat, sums_x, cnt, W1[:D], W1[D:],
                         b1.reshape(1, -1), batch_col)

    u2cat = _tc_prep2(h1, batch_col, degcat, sums_h1, cnt, W2[:256], W2[256:])
    S2cat = _make_conv(N, E, 64, False)(u2cat.reshape(2 * N, 64), idx2,
                                        dst_rows, jnp.zeros((NPT, 64), F32))
    WoAB = jnp.concatenate([Wo[:128], Wo[128:256]], axis=1)
    Acat, Bcat = _tc_h2p(S2cat, u2cat, degcat, b2.reshape(1, -1), WoAB)

    g = _make_edge_gather(N, E, 64)(Acat.reshape(2 * N, 64),
                                    Bcat.reshape(2 * N, 64), idx2, dst2)
    out_t = _tc_final(g, edge_attr, We, be.reshape(1, -1), Wo[256:],
                      bo.reshape(1, -1), Wf, bf.reshape(1, -1))
    return out_t.T
